# baseline (speedup 1.0000x reference)
import functools

import jax
import jax.numpy as jnp
from jax import lax
from jax.experimental import pallas as pl
from jax.experimental.pallas import tpu as pltpu
from jax.experimental.pallas import tpu_sc as plsc

BATCH = 16384
D = 64
NC = 2
NS = 16
NW = NC * NS
BPW = BATCH // NW
CHUNK = 128
NCHUNK = BPW // CHUNK


def _sc_gather(user, item, U_mf, I_mf, U_mlp, I_mlp):
    mesh = plsc.VectorSubcoreMesh(
        core_axis_name="c", subcore_axis_name="s",
        num_cores=NC, num_subcores=NS)

    @functools.partial(
        pl.kernel,
        out_type=(
            jax.ShapeDtypeStruct((BATCH, D), jnp.float32),
            jax.ShapeDtypeStruct((BATCH, D), jnp.float32),
            jax.ShapeDtypeStruct((BATCH, D), jnp.float32),
        ),
        mesh=mesh,
        compiler_params=pltpu.CompilerParams(use_tc_tiling_on_sc=False),
        scratch_types=(
            pltpu.VMEM((BPW,), jnp.int32),
            pltpu.VMEM((BPW,), jnp.int32),
            pltpu.VMEM((BPW, D), jnp.float32),
            pltpu.VMEM((BPW, D), jnp.float32),
            pltpu.SemaphoreType.DMA,
        ),
    )
    def k(user_h, item_h, umf_h, imf_h, umlp_h, imlp_h,
          out_u, out_i, out_mf, idx_u, idx_i, buf_a, buf_b, sem):
        wid = lax.axis_index("s") * NC + lax.axis_index("c")
        base = wid * BPW
        pltpu.sync_copy(user_h.at[pl.ds(base, BPW)], idx_u)
        pltpu.sync_copy(item_h.at[pl.ds(base, BPW)], idx_i)

        def gather(table_h, idx_v, buf):
            cps = [
                pltpu.async_copy(
                    table_h.at[idx_v.at[pl.ds(j * CHUNK, CHUNK)]],
                    buf.at[pl.ds(j * CHUNK, CHUNK)], sem)
                for j in range(NCHUNK)
            ]
            for cp in cps:
                cp.wait()

        gather(umlp_h, idx_u, buf_a)
        pltpu.sync_copy(buf_a, out_u.at[pl.ds(base, BPW)])
        gather(imlp_h, idx_i, buf_a)
        pltpu.sync_copy(buf_a, out_i.at[pl.ds(base, BPW)])

        gather(umf_h, idx_u, buf_a)
        gather(imf_h, idx_i, buf_b)

        def body(r, carry):
            for c in range(D // 16):
                s = pl.ds(c * 16, 16)
                buf_a[r, s] = buf_a[r, s] * buf_b[r, s]
            return carry
        lax.fori_loop(0, BPW, body, 0)
        pltpu.sync_copy(buf_a, out_mf.at[pl.ds(base, BPW)])

    return k(user, item, U_mf, I_mf, U_mlp, I_mlp)


def _tc_mlp(ug, ig, mf, W1u, W1i, b1, W2, b2, W3, b3, Wph, Wpm, bp):
    BT = 2048
    grid = (BATCH // BT,)

    def body(ug_r, ig_r, mf_r, w1u_r, w1i_r, b1_r, w2_r, b2_r, w3_r, b3_r,
             wph_r, wpm_r, bp_r, out_r):
        f32 = jnp.float32
        h = jnp.maximum(
            jnp.dot(ug_r[...], w1u_r[...], preferred_element_type=f32)
            + jnp.dot(ig_r[...], w1i_r[...], preferred_element_type=f32)
            + b1_r[...], 0.0)
        h = jnp.maximum(jnp.dot(h, w2_r[...], preferred_element_type=f32)
                        + b2_r[...], 0.0)
        h = jnp.maximum(jnp.dot(h, w3_r[...], preferred_element_type=f32)
                        + b3_r[...], 0.0)
        pred = (jnp.dot(h, wph_r[...], preferred_element_type=f32)
                + jnp.dot(mf_r[...], wpm_r[...], preferred_element_type=f32)
                + bp_r[0, 0])
        out_r[...] = jax.nn.sigmoid(pred)

    def full(shape):
        return pl.BlockSpec(shape, lambda i: (0, 0))

    return pl.pallas_call(
        body,
        grid=grid,
        in_specs=[
            pl.BlockSpec((BT, D), lambda i: (i, 0)),
            pl.BlockSpec((BT, D), lambda i: (i, 0)),
            pl.BlockSpec((BT, D), lambda i: (i, 0)),
            full((D, 64)), full((D, 64)), full((1, 64)),
            full((64, 32)), full((1, 32)),
            full((32, 16)), full((1, 16)),
            full((16, 1)), full((D, 1)), full((1, 1)),
        ],
        out_specs=pl.BlockSpec((BT, 1), lambda i: (i, 0)),
        out_shape=jax.ShapeDtypeStruct((BATCH, 1), jnp.float32),
    )(ug, ig, mf, W1u, W1i, b1, W2, b2, W3, b3, Wph, Wpm, bp)


def kernel(user, item, U_mf, I_mf, U_mlp, I_mlp,
           W1, b1, W2, b2, W3, b3, Wp, bp):
    ug, ig, mf = _sc_gather(user, item, U_mf, I_mf, U_mlp, I_mlp)
    pred = _tc_mlp(
        ug, ig, mf,
        W1[:D], W1[D:], b1[None, :],
        W2, b2[None, :], W3, b3[None, :],
        Wp[:16], Wp[16:], bp.reshape(1, 1))
    return pred[:, 0]
